# Initial kernel scaffold; baseline (speedup 1.0000x reference)
import jax
import jax.numpy as jnp
from jax.experimental import pallas as pl


def kernel(heads, embeds, tails, W):
    raise NotImplementedError("write your pallas kernel here")


# speedup vs baseline: 1.2770x; 1.2770x over previous
import functools

import jax
import jax.numpy as jnp
from jax import lax
from jax.experimental import pallas as pl
from jax.experimental.pallas import tpu as pltpu
from jax.experimental.pallas import tpu_sc as plsc

NUM_NEG = 20
R = NUM_NEG + 1
NW = 32
SUB = 128
SUBS_PER_CHUNK = 7
CHUNK = SUB * SUBS_PER_CHUNK


def _sc_gather_body(rows_per_w, n_chunks, W_hbm, idx_hbm, out_hbm,
                    idx_v, rows_v, sem):
    wid = lax.axis_index("s") * 2 + lax.axis_index("c")

    def chunk_body(ci, carry):
        base = wid * rows_per_w + ci * CHUNK
        pltpu.sync_copy(idx_hbm.at[pl.ds(base, CHUNK)], idx_v)
        copies = []
        for j in range(SUBS_PER_CHUNK):
            copies.append(pltpu.async_copy(
                W_hbm.at[idx_v.at[pl.ds(j * SUB, SUB)]],
                rows_v.at[pl.ds(j * SUB, SUB)],
                sem))
        for c in copies:
            c.wait()
        pltpu.sync_copy(rows_v, out_hbm.at[pl.ds(base, CHUNK)])
        return carry

    lax.fori_loop(0, n_chunks, chunk_body, 0)


def _sc_gather(W, idx):
    n_rows = idx.shape[0]
    d = W.shape[1]
    rows_per_w = n_rows // NW
    n_chunks = rows_per_w // CHUNK
    mesh = plsc.VectorSubcoreMesh(core_axis_name="c", subcore_axis_name="s")
    return pl.kernel(
        functools.partial(_sc_gather_body, rows_per_w, n_chunks),
        out_type=jax.ShapeDtypeStruct((n_rows, d), jnp.float32),
        mesh=mesh,
        scratch_types=[
            pltpu.VMEM((CHUNK,), jnp.int32),
            pltpu.VMEM((CHUNK, d), jnp.float32),
            pltpu.SemaphoreType.DMA,
        ],
        compiler_params=pltpu.CompilerParams(use_tc_tiling_on_sc=False),
    )(W, idx)


def _tc_body(num_heads, g_ref, e_ref, o_ref):
    i = pl.program_id(0)
    g = g_ref[...]
    e = e_ref[...]
    acc = None
    for r in range(R):
        s = jnp.sum(g[:, r * 32:(r + 1) * 32] * e, axis=1)
        sp = jnp.maximum(s, 0.0) + jnp.log1p(jnp.exp(-jnp.abs(s)))
        contrib = (s - sp) if r == 0 else (-sp)
        acc = contrib if acc is None else acc + contrib
    total = jnp.sum(acc) * (-1.0 / num_heads)

    @pl.when(i == 0)
    def _():
        o_ref[0, 0] = 0.0

    o_ref[0, 0] += total


def _tc_reduce(G2, embeds):
    num_heads = embeds.shape[0]
    blk = 256
    grid = num_heads // blk
    return pl.pallas_call(
        functools.partial(_tc_body, num_heads),
        grid=(grid,),
        in_specs=[
            pl.BlockSpec((blk, R * 32), lambda i: (i, 0)),
            pl.BlockSpec((blk, 32), lambda i: (i, 0)),
        ],
        out_specs=pl.BlockSpec(memory_space=pltpu.SMEM),
        out_shape=jax.ShapeDtypeStruct((1, 1), jnp.float32),
    )(G2, embeds)


def kernel(heads, embeds, tails, W):
    num_heads = heads.shape[0]
    num_nodes = W.shape[0]
    neg_key = jax.random.key(12345)
    neg_tails = jax.random.randint(
        neg_key, (num_heads * NUM_NEG,), 0, num_nodes, dtype=jnp.int32)
    neg_tails = neg_tails.reshape(num_heads, NUM_NEG)
    idx = jnp.concatenate([tails[:, None], neg_tails], axis=1).reshape(-1)
    G = _sc_gather(W, idx)
    G2 = G.reshape(num_heads, R * 32)
    out = _tc_reduce(G2, embeds)
    return out[0, 0]
